# baseline (speedup 1.0000x reference)
import jax
import jax.numpy as jnp
from jax import lax
from jax.experimental import pallas as pl
from jax.experimental.pallas import tpu as pltpu
from jax.experimental.pallas import tpu_sc as plsc

E = 8
N = 4096
D = 1024
P = 4

BA = 512
BC = 256
NSC = 2048
NT = N - NSC
TB = 4


def _logits_body(x_ref, w_ref, b_ref, perm_ref, lt_ref, permw_ref):
    lt = lax.dot_general(
        w_ref[...], x_ref[...],
        dimension_numbers=(((1,), (1,)), ((), ())),
        preferred_element_type=jnp.float32,
    )
    lt_ref[...] = lt + b_ref[...]
    permw_ref[...] = jnp.mean(perm_ref[...], axis=0)


def _gate_sc_body(lt_hbm, gates_hbm, lg_v, gates_v):
    info = plsc.get_sparse_core_info()
    nc, ns, nl = info.num_cores, info.num_subcores, info.num_lanes
    tpw = N // (nc * ns)
    wid = lax.axis_index("s") * nc + lax.axis_index("c")
    base = wid * tpw
    pltpu.sync_copy(lt_hbm.at[:, pl.ds(base, tpw)], lg_v)
    for g in range(tpw // nl):
        l = [lg_v[e, pl.ds(g * nl, nl)] for e in range(E)]
        best = l[0]
        i1 = jnp.zeros((nl,), jnp.int32)
        for e in range(1, E):
            c = l[e] > best
            best = jnp.where(c, l[e], best)
            i1 = jnp.where(c, e, i1)
        best2 = jnp.full((nl,), -jnp.inf, jnp.float32)
        i2 = jnp.zeros((nl,), jnp.int32)
        for e in range(E):
            c = (i1 != e) & (l[e] > best2)
            best2 = jnp.where(c, l[e], best2)
            i2 = jnp.where(c, e, i2)
        p = []
        for e in range(E):
            sel = ((i1 == e) | (i2 == e)) & (l[e] != 0.0)
            p.append(jnp.where(sel, jnp.exp(l[e] - best), 0.0))
        denom = p[0]
        for e in range(1, E):
            denom = denom + p[e]
        inv = 1.0 / denom
        for e in range(E):
            gates_v[e, pl.ds(g * nl, nl)] = p[e] * inv
    pltpu.sync_copy(gates_v, gates_hbm.at[:, pl.ds(base, tpw)])


def _gpn_from_gates(g_t, permw):
    g = jnp.transpose(g_t)
    gp = lax.dot_general(
        g, permw,
        dimension_numbers=(((1,), (0,)), ((), ())),
        preferred_element_type=jnp.float32,
    )
    return gp / jnp.sum(gp, axis=1, keepdims=True)


def _gexp_body(gates_ref, permw_ref, gexp_ref):
    gpn = _gpn_from_gates(gates_ref[:, NT:], permw_ref[...])
    row = lax.broadcasted_iota(jnp.int32, (E, 128), 0)
    col = lax.broadcasted_iota(jnp.int32, (E, 128), 1)
    rep = jnp.where(col // 16 == row, 1.0, 0.0)
    gexp_ref[...] = lax.dot_general(
        gpn, rep,
        dimension_numbers=(((1,), (0,)), ((), ())),
        preferred_element_type=jnp.float32,
    )


def _combine_body(gates_ref, h_ref, permw_ref, y_ref, soft_ref, hard_ref):
    pid = pl.program_id(0)
    nblk = pl.num_programs(0)
    permw = permw_ref[...]
    gpn = _gpn_from_gates(gates_ref[:, pl.ds(pid * BC, BC)], permw)
    acc = gpn[:, 0:1] * h_ref[0]
    for e in range(1, E):
        acc = acc + gpn[:, e:e + 1] * h_ref[e]
    y_ref[...] = acc
    psoft = jnp.sum(gpn, axis=0, keepdims=True)
    phard = jnp.sum(jnp.where(gpn < 1e-5, 0.0, 1.0), axis=0, keepdims=True)

    @pl.when(pid == 0)
    def _init():
        soft_ref[...] = psoft
        hard_ref[...] = phard

    @pl.when(pid != 0)
    def _acc():
        soft_ref[...] += psoft
        hard_ref[...] += phard

    @pl.when(pid == nblk - 1)
    def _fin():
        ssoft = soft_ref[...]
        shard = hard_ref[...]
        for j in range(NT // BC, N // BC):
            gpn_j = _gpn_from_gates(gates_ref[:, pl.ds(j * BC, BC)], permw)
            ssoft += jnp.sum(gpn_j, axis=0, keepdims=True)
            shard += jnp.sum(jnp.where(gpn_j < 1e-5, 0.0, 1.0), axis=0,
                             keepdims=True)
        soft_ref[...] = ssoft * (1.0 / N)
        hard_ref[...] = shard * (1.0 / N)


def _sc_combine_body(h_hbm, gexp_hbm, ysc_hbm, gexp_v, hb, yv,
                     hsem0, hsem1, ysem0, ysem1):
    hsem = [hsem0, hsem1]
    ysem = [ysem0, ysem1]
    info = plsc.get_sparse_core_info()
    nc, ns, nl = info.num_cores, info.num_subcores, info.num_lanes
    nw = nc * ns
    tw = NSC // nw
    nsb = tw // TB
    wid = lax.axis_index("s") * nc + lax.axis_index("c")
    wbase = wid * tw
    pltpu.sync_copy(gexp_hbm.at[pl.ds(wbase, tw), :], gexp_v)

    def in_slice(s, e):
        return h_hbm.at[e, pl.ds(NT + wbase + s * TB, TB), :]

    def start_in(s, b):
        for e in range(E):
            pltpu.async_copy(in_slice(s, e), hb.at[b, e], hsem[b])

    def wait_in(s, b):
        for e in range(E):
            pltpu.make_async_copy(in_slice(s, e), hb.at[b, e], hsem[b]).wait()

    def out_slice(s):
        return ysc_hbm.at[pl.ds(wbase + s * TB, TB), :]

    start_in(0, 0)
    start_in(1, 1)

    def step(i, _):
        for b in range(2):
            s = i * 2 + b
            wait_in(s, b)

            @pl.when(s >= 2)
            def _drain_y():
                pltpu.make_async_copy(out_slice(s - 2), yv.at[b],
                                      ysem[b]).wait()

            for t in range(TB):
                tloc = s * TB + t
                spl = [gexp_v[tloc, pl.ds(e * nl, nl)] for e in range(E)]

                def chunk(c4, _c):
                    for k in range(4):
                        off = c4 * (4 * nl) + k * nl
                        a = spl[0] * hb[b, 0, t, pl.ds(off, nl)]
                        for e in range(1, E):
                            a = a + spl[e] * hb[b, e, t, pl.ds(off, nl)]
                        yv[b, t, pl.ds(off, nl)] = a
                    return 0

                lax.fori_loop(0, D // (4 * nl), chunk, 0)

            pltpu.async_copy(yv.at[b], out_slice(s), ysem[b])

            @pl.when(s + 2 < nsb)
            def _next_in():
                start_in(s + 2, b)
        return 0

    lax.fori_loop(0, nsb // 2, step, 0)
    for b in range(2):
        pltpu.make_async_copy(out_slice(nsb - 2 + b), yv.at[b],
                              ysem[b]).wait()


def _merge_body(yfull_ref, ysc_ref, y_ref):
    y_ref[...] = ysc_ref[...]


def kernel(h, x, permutation_weights, W, b):
    b2 = b.reshape(E, 1)

    logits_t, permw = pl.pallas_call(
        _logits_body,
        grid=(N // BA,),
        in_specs=[
            pl.BlockSpec((BA, D), lambda i: (i, 0)),
            pl.BlockSpec((E, D), lambda i: (0, 0)),
            pl.BlockSpec((E, 1), lambda i: (0, 0)),
            pl.BlockSpec((P, E, E), lambda i: (0, 0, 0)),
        ],
        out_specs=[
            pl.BlockSpec((E, BA), lambda i: (0, i)),
            pl.BlockSpec((E, E), lambda i: (0, 0)),
        ],
        out_shape=[
            jax.ShapeDtypeStruct((E, N), jnp.float32),
            jax.ShapeDtypeStruct((E, E), jnp.float32),
        ],
    )(x, W, b2, permutation_weights)

    info = plsc.get_sparse_core_info()
    nw = info.num_cores * info.num_subcores
    tpw = N // nw
    gates = pl.kernel(
        _gate_sc_body,
        out_type=jax.ShapeDtypeStruct((E, N), jnp.float32),
        mesh=plsc.VectorSubcoreMesh(core_axis_name="c", subcore_axis_name="s"),
        scratch_types=[
            pltpu.VMEM((E, tpw), jnp.float32),
            pltpu.VMEM((E, tpw), jnp.float32),
        ],
    )(logits_t)

    gexp = pl.pallas_call(
        _gexp_body,
        in_specs=[
            pl.BlockSpec((E, N), lambda: (0, 0)),
            pl.BlockSpec((E, E), lambda: (0, 0)),
        ],
        out_specs=pl.BlockSpec((NSC, 128), lambda: (0, 0)),
        out_shape=jax.ShapeDtypeStruct((NSC, 128), jnp.float32),
    )(gates, permw)

    tw = NSC // nw
    ysc = pl.kernel(
        _sc_combine_body,
        out_type=jax.ShapeDtypeStruct((NSC, D), jnp.float32),
        mesh=plsc.VectorSubcoreMesh(core_axis_name="c", subcore_axis_name="s"),
        scratch_types=[
            pltpu.VMEM((tw, 128), jnp.float32),
            pltpu.VMEM((2, E, TB, D), jnp.float32),
            pltpu.VMEM((2, TB, D), jnp.float32),
            pltpu.SemaphoreType.DMA,
            pltpu.SemaphoreType.DMA,
            pltpu.SemaphoreType.DMA,
            pltpu.SemaphoreType.DMA,
        ],
    )(h, gexp)

    y_part, soft, hard = pl.pallas_call(
        _combine_body,
        grid=(NT // BC,),
        in_specs=[
            pl.BlockSpec((E, N), lambda i: (0, 0)),
            pl.BlockSpec((E, BC, D), lambda i: (0, i, 0)),
            pl.BlockSpec((E, E), lambda i: (0, 0)),
        ],
        out_specs=[
            pl.BlockSpec((BC, D), lambda i: (i, 0)),
            pl.BlockSpec((1, E), lambda i: (0, 0)),
            pl.BlockSpec((1, E), lambda i: (0, 0)),
        ],
        out_shape=[
            jax.ShapeDtypeStruct((N, D), jnp.float32),
            jax.ShapeDtypeStruct((1, E), jnp.float32),
            jax.ShapeDtypeStruct((1, E), jnp.float32),
        ],
    )(gates, h, permw)

    y = pl.pallas_call(
        _merge_body,
        grid=(NSC // BC,),
        in_specs=[
            pl.BlockSpec((BC, D), lambda i: (0, 0)),
            pl.BlockSpec((BC, D), lambda i: (i, 0)),
        ],
        out_specs=pl.BlockSpec((BC, D), lambda i: (NT // BC + i, 0)),
        out_shape=jax.ShapeDtypeStruct((N, D), jnp.float32),
        input_output_aliases={0: 0},
    )(y_part, ysc)

    return (y, soft.reshape(E, 1), hard.reshape(E, 1))


# speedup vs baseline: 1.1733x; 1.1733x over previous
import jax
import jax.numpy as jnp
from jax import lax
from jax.experimental import pallas as pl
from jax.experimental.pallas import tpu as pltpu
from jax.experimental.pallas import tpu_sc as plsc

E = 8
N = 4096
D = 1024
P = 4

BA = 512
BC = 256
NSC = 1536
NT = N - NSC
TB = 4


def _logits_body(x_ref, w_ref, b_ref, perm_ref, lt_ref, permw_ref):
    lt = lax.dot_general(
        w_ref[...], x_ref[...],
        dimension_numbers=(((1,), (1,)), ((), ())),
        preferred_element_type=jnp.float32,
    )
    lt_ref[...] = lt + b_ref[...]
    permw_ref[...] = jnp.mean(perm_ref[...], axis=0)


def _gate_sc_body(lt_hbm, gates_hbm, lg_v, gates_v):
    info = plsc.get_sparse_core_info()
    nc, ns, nl = info.num_cores, info.num_subcores, info.num_lanes
    tpw = N // (nc * ns)
    wid = lax.axis_index("s") * nc + lax.axis_index("c")
    base = wid * tpw
    pltpu.sync_copy(lt_hbm.at[:, pl.ds(base, tpw)], lg_v)
    for g in range(tpw // nl):
        l = [lg_v[e, pl.ds(g * nl, nl)] for e in range(E)]
        best = l[0]
        i1 = jnp.zeros((nl,), jnp.int32)
        for e in range(1, E):
            c = l[e] > best
            best = jnp.where(c, l[e], best)
            i1 = jnp.where(c, e, i1)
        best2 = jnp.full((nl,), -jnp.inf, jnp.float32)
        i2 = jnp.zeros((nl,), jnp.int32)
        for e in range(E):
            c = (i1 != e) & (l[e] > best2)
            best2 = jnp.where(c, l[e], best2)
            i2 = jnp.where(c, e, i2)
        p = []
        for e in range(E):
            sel = ((i1 == e) | (i2 == e)) & (l[e] != 0.0)
            p.append(jnp.where(sel, jnp.exp(l[e] - best), 0.0))
        denom = p[0]
        for e in range(1, E):
            denom = denom + p[e]
        inv = 1.0 / denom
        for e in range(E):
            gates_v[e, pl.ds(g * nl, nl)] = p[e] * inv
    pltpu.sync_copy(gates_v, gates_hbm.at[:, pl.ds(base, tpw)])


def _gpn_from_gates(g_t, permw):
    g = jnp.transpose(g_t)
    gp = lax.dot_general(
        g, permw,
        dimension_numbers=(((1,), (0,)), ((), ())),
        preferred_element_type=jnp.float32,
    )
    return gp / jnp.sum(gp, axis=1, keepdims=True)


def _gexp_body(gates_ref, permw_ref, gexp_ref):
    gpn = _gpn_from_gates(gates_ref[:, NT:], permw_ref[...])
    row = lax.broadcasted_iota(jnp.int32, (E, 128), 0)
    col = lax.broadcasted_iota(jnp.int32, (E, 128), 1)
    rep = jnp.where(col // 16 == row, 1.0, 0.0)
    gexp_ref[...] = lax.dot_general(
        gpn, rep,
        dimension_numbers=(((1,), (0,)), ((), ())),
        preferred_element_type=jnp.float32,
    )


def _combine_body(gates_ref, h_ref, permw_ref, y_ref, soft_ref, hard_ref):
    pid = pl.program_id(0)
    nblk = pl.num_programs(0)
    permw = permw_ref[...]
    gpn = _gpn_from_gates(gates_ref[:, pl.ds(pid * BC, BC)], permw)
    acc = gpn[:, 0:1] * h_ref[0]
    for e in range(1, E):
        acc = acc + gpn[:, e:e + 1] * h_ref[e]
    y_ref[...] = acc
    psoft = jnp.sum(gpn, axis=0, keepdims=True)
    phard = jnp.sum(jnp.where(gpn < 1e-5, 0.0, 1.0), axis=0, keepdims=True)

    @pl.when(pid == 0)
    def _init():
        soft_ref[...] = psoft
        hard_ref[...] = phard

    @pl.when(pid != 0)
    def _acc():
        soft_ref[...] += psoft
        hard_ref[...] += phard

    @pl.when(pid == nblk - 1)
    def _fin():
        ssoft = soft_ref[...]
        shard = hard_ref[...]
        for j in range(NT // BC, N // BC):
            gpn_j = _gpn_from_gates(gates_ref[:, pl.ds(j * BC, BC)], permw)
            ssoft += jnp.sum(gpn_j, axis=0, keepdims=True)
            shard += jnp.sum(jnp.where(gpn_j < 1e-5, 0.0, 1.0), axis=0,
                             keepdims=True)
        soft_ref[...] = ssoft * (1.0 / N)
        hard_ref[...] = shard * (1.0 / N)


def _sc_combine_body(h_hbm, gexp_hbm, ysc_hbm, gexp_v, hb, yv,
                     hsem0, hsem1, ysem0, ysem1):
    hsem = [hsem0, hsem1]
    ysem = [ysem0, ysem1]
    info = plsc.get_sparse_core_info()
    nc, ns, nl = info.num_cores, info.num_subcores, info.num_lanes
    nw = nc * ns
    tw = NSC // nw
    nsb = tw // TB
    wid = lax.axis_index("s") * nc + lax.axis_index("c")
    wbase = wid * tw
    pltpu.sync_copy(gexp_hbm.at[pl.ds(wbase, tw), :], gexp_v)

    def in_slice(s):
        return h_hbm.at[:, pl.ds(NT + wbase + s * TB, TB), :]

    def start_in(s, b):
        pltpu.async_copy(in_slice(s), hb.at[b], hsem[b])

    def wait_in(s, b):
        pltpu.make_async_copy(in_slice(s), hb.at[b], hsem[b]).wait()

    def out_slice(s):
        return ysc_hbm.at[pl.ds(wbase + s * TB, TB), :]

    start_in(0, 0)
    start_in(1, 1)

    def step(i, _):
        for b in range(2):
            s = i * 2 + b
            wait_in(s, b)

            @pl.when(s >= 2)
            def _drain_y():
                pltpu.make_async_copy(out_slice(s - 2), yv.at[b],
                                      ysem[b]).wait()

            for t in range(TB):
                tloc = s * TB + t
                spl = [gexp_v[tloc, pl.ds(e * nl, nl)] for e in range(E)]

                def chunk(c4, _c):
                    for k in range(4):
                        off = c4 * (4 * nl) + k * nl
                        a = spl[0] * hb[b, 0, t, pl.ds(off, nl)]
                        for e in range(1, E):
                            a = a + spl[e] * hb[b, e, t, pl.ds(off, nl)]
                        yv[b, t, pl.ds(off, nl)] = a
                    return 0

                lax.fori_loop(0, D // (4 * nl), chunk, 0)

            pltpu.async_copy(yv.at[b], out_slice(s), ysem[b])

            @pl.when(s + 2 < nsb)
            def _next_in():
                start_in(s + 2, b)
        return 0

    lax.fori_loop(0, nsb // 2, step, 0)
    for b in range(2):
        pltpu.make_async_copy(out_slice(nsb - 2 + b), yv.at[b],
                              ysem[b]).wait()


def _merge_body(yfull_ref, ysc_ref, y_ref):
    y_ref[...] = ysc_ref[...]


def kernel(h, x, permutation_weights, W, b):
    b2 = b.reshape(E, 1)

    logits_t, permw = pl.pallas_call(
        _logits_body,
        grid=(N // BA,),
        in_specs=[
            pl.BlockSpec((BA, D), lambda i: (i, 0)),
            pl.BlockSpec((E, D), lambda i: (0, 0)),
            pl.BlockSpec((E, 1), lambda i: (0, 0)),
            pl.BlockSpec((P, E, E), lambda i: (0, 0, 0)),
        ],
        out_specs=[
            pl.BlockSpec((E, BA), lambda i: (0, i)),
            pl.BlockSpec((E, E), lambda i: (0, 0)),
        ],
        out_shape=[
            jax.ShapeDtypeStruct((E, N), jnp.float32),
            jax.ShapeDtypeStruct((E, E), jnp.float32),
        ],
    )(x, W, b2, permutation_weights)

    info = plsc.get_sparse_core_info()
    nw = info.num_cores * info.num_subcores
    tpw = N // nw
    gates = pl.kernel(
        _gate_sc_body,
        out_type=jax.ShapeDtypeStruct((E, N), jnp.float32),
        mesh=plsc.VectorSubcoreMesh(core_axis_name="c", subcore_axis_name="s"),
        scratch_types=[
            pltpu.VMEM((E, tpw), jnp.float32),
            pltpu.VMEM((E, tpw), jnp.float32),
        ],
    )(logits_t)

    gexp = pl.pallas_call(
        _gexp_body,
        in_specs=[
            pl.BlockSpec((E, N), lambda: (0, 0)),
            pl.BlockSpec((E, E), lambda: (0, 0)),
        ],
        out_specs=pl.BlockSpec((NSC, 128), lambda: (0, 0)),
        out_shape=jax.ShapeDtypeStruct((NSC, 128), jnp.float32),
    )(gates, permw)

    tw = NSC // nw
    ysc = pl.kernel(
        _sc_combine_body,
        out_type=jax.ShapeDtypeStruct((NSC, D), jnp.float32),
        mesh=plsc.VectorSubcoreMesh(core_axis_name="c", subcore_axis_name="s"),
        scratch_types=[
            pltpu.VMEM((tw, 128), jnp.float32),
            pltpu.VMEM((2, E, TB, D), jnp.float32),
            pltpu.VMEM((2, TB, D), jnp.float32),
            pltpu.SemaphoreType.DMA,
            pltpu.SemaphoreType.DMA,
            pltpu.SemaphoreType.DMA,
            pltpu.SemaphoreType.DMA,
        ],
    )(h, gexp)

    y_part, soft, hard = pl.pallas_call(
        _combine_body,
        grid=(NT // BC,),
        in_specs=[
            pl.BlockSpec((E, N), lambda i: (0, 0)),
            pl.BlockSpec((E, BC, D), lambda i: (0, i, 0)),
            pl.BlockSpec((E, E), lambda i: (0, 0)),
        ],
        out_specs=[
            pl.BlockSpec((BC, D), lambda i: (i, 0)),
            pl.BlockSpec((1, E), lambda i: (0, 0)),
            pl.BlockSpec((1, E), lambda i: (0, 0)),
        ],
        out_shape=[
            jax.ShapeDtypeStruct((N, D), jnp.float32),
            jax.ShapeDtypeStruct((1, E), jnp.float32),
            jax.ShapeDtypeStruct((1, E), jnp.float32),
        ],
    )(gates, h, permw)

    y = pl.pallas_call(
        _merge_body,
        grid=(NSC // BC,),
        in_specs=[
            pl.BlockSpec((BC, D), lambda i: (0, 0)),
            pl.BlockSpec((BC, D), lambda i: (i, 0)),
        ],
        out_specs=pl.BlockSpec((BC, D), lambda i: (NT // BC + i, 0)),
        out_shape=jax.ShapeDtypeStruct((N, D), jnp.float32),
        input_output_aliases={0: 0},
    )(y_part, ysc)

    return (y, soft.reshape(E, 1), hard.reshape(E, 1))


# speedup vs baseline: 1.2628x; 1.0763x over previous
import jax
import jax.numpy as jnp
from jax import lax
from jax.experimental import pallas as pl
from jax.experimental.pallas import tpu as pltpu
from jax.experimental.pallas import tpu_sc as plsc

E = 8
N = 4096
D = 1024
P = 4

BA = 512
BC = 256
NSC = 1280
NT = N - NSC
TB = 4


def _logits_body(x_ref, w_ref, b_ref, perm_ref, lt_ref, permw_ref):
    lt = lax.dot_general(
        w_ref[...], x_ref[...],
        dimension_numbers=(((1,), (1,)), ((), ())),
        preferred_element_type=jnp.float32,
    )
    lt_ref[...] = lt + b_ref[...]
    permw_ref[...] = jnp.mean(perm_ref[...], axis=0)


def _gate_sc_body(lt_hbm, gates_hbm, lg_v, gates_v):
    info = plsc.get_sparse_core_info()
    nc, ns, nl = info.num_cores, info.num_subcores, info.num_lanes
    tpw = N // (nc * ns)
    wid = lax.axis_index("s") * nc + lax.axis_index("c")
    base = wid * tpw
    pltpu.sync_copy(lt_hbm.at[:, pl.ds(base, tpw)], lg_v)
    for g in range(tpw // nl):
        l = [lg_v[e, pl.ds(g * nl, nl)] for e in range(E)]
        best = l[0]
        i1 = jnp.zeros((nl,), jnp.int32)
        for e in range(1, E):
            c = l[e] > best
            best = jnp.where(c, l[e], best)
            i1 = jnp.where(c, e, i1)
        best2 = jnp.full((nl,), -jnp.inf, jnp.float32)
        i2 = jnp.zeros((nl,), jnp.int32)
        for e in range(E):
            c = (i1 != e) & (l[e] > best2)
            best2 = jnp.where(c, l[e], best2)
            i2 = jnp.where(c, e, i2)
        p = []
        for e in range(E):
            sel = ((i1 == e) | (i2 == e)) & (l[e] != 0.0)
            p.append(jnp.where(sel, jnp.exp(l[e] - best), 0.0))
        denom = p[0]
        for e in range(1, E):
            denom = denom + p[e]
        inv = 1.0 / denom
        for e in range(E):
            gates_v[e, pl.ds(g * nl, nl)] = p[e] * inv
    pltpu.sync_copy(gates_v, gates_hbm.at[:, pl.ds(base, tpw)])


def _gpn_from_gates(g_t, permw):
    g = jnp.transpose(g_t)
    gp = lax.dot_general(
        g, permw,
        dimension_numbers=(((1,), (0,)), ((), ())),
        preferred_element_type=jnp.float32,
    )
    return gp / jnp.sum(gp, axis=1, keepdims=True)


def _gexp_body(gates_ref, permw_ref, gexp_ref):
    gpn = _gpn_from_gates(gates_ref[:, NT:], permw_ref[...])
    row = lax.broadcasted_iota(jnp.int32, (E, 128), 0)
    col = lax.broadcasted_iota(jnp.int32, (E, 128), 1)
    rep = jnp.where(col // 16 == row, 1.0, 0.0)
    gexp_ref[...] = lax.dot_general(
        gpn, rep,
        dimension_numbers=(((1,), (0,)), ((), ())),
        preferred_element_type=jnp.float32,
    )


def _combine_body(gates_ref, h_ref, permw_ref, y_ref, soft_ref, hard_ref):
    pid = pl.program_id(0)
    nblk = pl.num_programs(0)
    permw = permw_ref[...]
    gpn = _gpn_from_gates(gates_ref[:, pl.ds(pid * BC, BC)], permw)
    acc = gpn[:, 0:1] * h_ref[0]
    for e in range(1, E):
        acc = acc + gpn[:, e:e + 1] * h_ref[e]
    y_ref[...] = acc
    psoft = jnp.sum(gpn, axis=0, keepdims=True)
    phard = jnp.sum(jnp.where(gpn < 1e-5, 0.0, 1.0), axis=0, keepdims=True)

    @pl.when(pid == 0)
    def _init():
        soft_ref[...] = psoft
        hard_ref[...] = phard

    @pl.when(pid != 0)
    def _acc():
        soft_ref[...] += psoft
        hard_ref[...] += phard

    @pl.when(pid == nblk - 1)
    def _fin():
        ssoft = soft_ref[...]
        shard = hard_ref[...]
        for j in range(NT // BC, N // BC):
            gpn_j = _gpn_from_gates(gates_ref[:, pl.ds(j * BC, BC)], permw)
            ssoft += jnp.sum(gpn_j, axis=0, keepdims=True)
            shard += jnp.sum(jnp.where(gpn_j < 1e-5, 0.0, 1.0), axis=0,
                             keepdims=True)
        soft_ref[...] = ssoft * (1.0 / N)
        hard_ref[...] = shard * (1.0 / N)


def _sc_combine_body(h_hbm, gexp_hbm, ysc_hbm, gexp_v, hb, yv,
                     hsem0, hsem1, ysem0, ysem1):
    hsem = [hsem0, hsem1]
    ysem = [ysem0, ysem1]
    info = plsc.get_sparse_core_info()
    nc, ns, nl = info.num_cores, info.num_subcores, info.num_lanes
    nw = nc * ns
    tw = NSC // nw
    nsb = tw // TB
    wid = lax.axis_index("s") * nc + lax.axis_index("c")
    wbase = wid * tw
    pltpu.sync_copy(gexp_hbm.at[pl.ds(wbase, tw), :], gexp_v)

    def in_slice(s):
        return h_hbm.at[:, pl.ds(NT + wbase + s * TB, TB), :]

    def start_in(s, b):
        pltpu.async_copy(in_slice(s), hb.at[b], hsem[b])

    def wait_in(s, b):
        pltpu.make_async_copy(in_slice(s), hb.at[b], hsem[b]).wait()

    def out_slice(s):
        return ysc_hbm.at[pl.ds(wbase + s * TB, TB), :]

    start_in(0, 0)
    start_in(1, 1)

    def step(i, _):
        for b in range(2):
            s = i * 2 + b
            wait_in(s, b)

            @pl.when(s >= 2)
            def _drain_y():
                pltpu.make_async_copy(out_slice(s - 2), yv.at[b],
                                      ysem[b]).wait()

            for t in range(TB):
                tloc = s * TB + t
                spl = [gexp_v[tloc, pl.ds(e * nl, nl)] for e in range(E)]

                def chunk(c4, _c):
                    for k in range(4):
                        off = c4 * (4 * nl) + k * nl
                        a = spl[0] * hb[b, 0, t, pl.ds(off, nl)]
                        for e in range(1, E):
                            a = a + spl[e] * hb[b, e, t, pl.ds(off, nl)]
                        yv[b, t, pl.ds(off, nl)] = a
                    return 0

                lax.fori_loop(0, D // (4 * nl), chunk, 0)

            pltpu.async_copy(yv.at[b], out_slice(s), ysem[b])

            @pl.when(s + 2 < nsb)
            def _next_in():
                start_in(s + 2, b)
        return 0

    lax.fori_loop(0, nsb // 2, step, 0)
    for b in range(2):
        pltpu.make_async_copy(out_slice(nsb - 2 + b), yv.at[b],
                              ysem[b]).wait()


def _merge_body(yfull_ref, ysc_ref, y_ref):
    y_ref[...] = ysc_ref[...]


def kernel(h, x, permutation_weights, W, b):
    b2 = b.reshape(E, 1)

    logits_t, permw = pl.pallas_call(
        _logits_body,
        grid=(N // BA,),
        in_specs=[
            pl.BlockSpec((BA, D), lambda i: (i, 0)),
            pl.BlockSpec((E, D), lambda i: (0, 0)),
            pl.BlockSpec((E, 1), lambda i: (0, 0)),
            pl.BlockSpec((P, E, E), lambda i: (0, 0, 0)),
        ],
        out_specs=[
            pl.BlockSpec((E, BA), lambda i: (0, i)),
            pl.BlockSpec((E, E), lambda i: (0, 0)),
        ],
        out_shape=[
            jax.ShapeDtypeStruct((E, N), jnp.float32),
            jax.ShapeDtypeStruct((E, E), jnp.float32),
        ],
    )(x, W, b2, permutation_weights)

    info = plsc.get_sparse_core_info()
    nw = info.num_cores * info.num_subcores
    tpw = N // nw
    gates = pl.kernel(
        _gate_sc_body,
        out_type=jax.ShapeDtypeStruct((E, N), jnp.float32),
        mesh=plsc.VectorSubcoreMesh(core_axis_name="c", subcore_axis_name="s"),
        scratch_types=[
            pltpu.VMEM((E, tpw), jnp.float32),
            pltpu.VMEM((E, tpw), jnp.float32),
        ],
    )(logits_t)

    gexp = pl.pallas_call(
        _gexp_body,
        in_specs=[
            pl.BlockSpec((E, N), lambda: (0, 0)),
            pl.BlockSpec((E, E), lambda: (0, 0)),
        ],
        out_specs=pl.BlockSpec((NSC, 128), lambda: (0, 0)),
        out_shape=jax.ShapeDtypeStruct((NSC, 128), jnp.float32),
    )(gates, permw)

    tw = NSC // nw
    ysc = pl.kernel(
        _sc_combine_body,
        out_type=jax.ShapeDtypeStruct((NSC, D), jnp.float32),
        mesh=plsc.VectorSubcoreMesh(core_axis_name="c", subcore_axis_name="s"),
        scratch_types=[
            pltpu.VMEM((tw, 128), jnp.float32),
            pltpu.VMEM((2, E, TB, D), jnp.float32),
            pltpu.VMEM((2, TB, D), jnp.float32),
            pltpu.SemaphoreType.DMA,
            pltpu.SemaphoreType.DMA,
            pltpu.SemaphoreType.DMA,
            pltpu.SemaphoreType.DMA,
        ],
    )(h, gexp)

    y_part, soft, hard = pl.pallas_call(
        _combine_body,
        grid=(NT // BC,),
        in_specs=[
            pl.BlockSpec((E, N), lambda i: (0, 0)),
            pl.BlockSpec((E, BC, D), lambda i: (0, i, 0)),
            pl.BlockSpec((E, E), lambda i: (0, 0)),
        ],
        out_specs=[
            pl.BlockSpec((BC, D), lambda i: (i, 0)),
            pl.BlockSpec((1, E), lambda i: (0, 0)),
            pl.BlockSpec((1, E), lambda i: (0, 0)),
        ],
        out_shape=[
            jax.ShapeDtypeStruct((N, D), jnp.float32),
            jax.ShapeDtypeStruct((1, E), jnp.float32),
            jax.ShapeDtypeStruct((1, E), jnp.float32),
        ],
    )(gates, h, permw)

    y = pl.pallas_call(
        _merge_body,
        grid=(NSC // BC,),
        in_specs=[
            pl.BlockSpec((BC, D), lambda i: (0, 0)),
            pl.BlockSpec((BC, D), lambda i: (i, 0)),
        ],
        out_specs=pl.BlockSpec((BC, D), lambda i: (NT // BC + i, 0)),
        out_shape=jax.ShapeDtypeStruct((N, D), jnp.float32),
        input_output_aliases={0: 0},
    )(y_part, ysc)

    return (y, soft.reshape(E, 1), hard.reshape(E, 1))


# speedup vs baseline: 1.3338x; 1.0562x over previous
import jax
import jax.numpy as jnp
from jax import lax
from jax.experimental import pallas as pl
from jax.experimental.pallas import tpu as pltpu
from jax.experimental.pallas import tpu_sc as plsc

E = 8
N = 4096
D = 1024
P = 4

BA = 512
BC = 256
NSC = 512
NT = N - NSC
TB = 4


def _logits_body(x_ref, w_ref, b_ref, perm_ref, lt_ref, permw_ref):
    lt = lax.dot_general(
        w_ref[...], x_ref[...],
        dimension_numbers=(((1,), (1,)), ((), ())),
        preferred_element_type=jnp.float32,
    )
    lt_ref[...] = lt + b_ref[...]
    permw_ref[...] = jnp.mean(perm_ref[...], axis=0)


def _gate_sc_body(lt_hbm, gates_hbm, lg_v, gates_v):
    info = plsc.get_sparse_core_info()
    nc, ns, nl = info.num_cores, info.num_subcores, info.num_lanes
    tpw = N // (nc * ns)
    wid = lax.axis_index("s") * nc + lax.axis_index("c")
    base = wid * tpw
    pltpu.sync_copy(lt_hbm.at[:, pl.ds(base, tpw)], lg_v)
    for g in range(tpw // nl):
        l = [lg_v[e, pl.ds(g * nl, nl)] for e in range(E)]
        best = l[0]
        i1 = jnp.zeros((nl,), jnp.int32)
        for e in range(1, E):
            c = l[e] > best
            best = jnp.where(c, l[e], best)
            i1 = jnp.where(c, e, i1)
        best2 = jnp.full((nl,), -jnp.inf, jnp.float32)
        i2 = jnp.zeros((nl,), jnp.int32)
        for e in range(E):
            c = (i1 != e) & (l[e] > best2)
            best2 = jnp.where(c, l[e], best2)
            i2 = jnp.where(c, e, i2)
        p = []
        for e in range(E):
            sel = ((i1 == e) | (i2 == e)) & (l[e] != 0.0)
            p.append(jnp.where(sel, jnp.exp(l[e] - best), 0.0))
        denom = p[0]
        for e in range(1, E):
            denom = denom + p[e]
        inv = 1.0 / denom
        for e in range(E):
            gates_v[e, pl.ds(g * nl, nl)] = p[e] * inv
    pltpu.sync_copy(gates_v, gates_hbm.at[:, pl.ds(base, tpw)])


def _gpn_from_gates(g_t, permw):
    g = jnp.transpose(g_t)
    gp = lax.dot_general(
        g, permw,
        dimension_numbers=(((1,), (0,)), ((), ())),
        preferred_element_type=jnp.float32,
    )
    return gp / jnp.sum(gp, axis=1, keepdims=True)


def _gexp_body(gates_ref, permw_ref, gexp_ref):
    gpn = _gpn_from_gates(gates_ref[:, NT:], permw_ref[...])
    row = lax.broadcasted_iota(jnp.int32, (E, 128), 0)
    col = lax.broadcasted_iota(jnp.int32, (E, 128), 1)
    rep = jnp.where(col // 16 == row, 1.0, 0.0)
    gexp_ref[...] = lax.dot_general(
        gpn, rep,
        dimension_numbers=(((1,), (0,)), ((), ())),
        preferred_element_type=jnp.float32,
    )


def _combine_body(gates_ref, h_ref, permw_ref, y_ref, soft_ref, hard_ref):
    pid = pl.program_id(0)
    nblk = pl.num_programs(0)
    permw = permw_ref[...]
    gpn = _gpn_from_gates(gates_ref[:, pl.ds(pid * BC, BC)], permw)
    acc = gpn[:, 0:1] * h_ref[0]
    for e in range(1, E):
        acc = acc + gpn[:, e:e + 1] * h_ref[e]
    y_ref[...] = acc
    psoft = jnp.sum(gpn, axis=0, keepdims=True)
    phard = jnp.sum(jnp.where(gpn < 1e-5, 0.0, 1.0), axis=0, keepdims=True)

    @pl.when(pid == 0)
    def _init():
        soft_ref[...] = psoft
        hard_ref[...] = phard

    @pl.when(pid != 0)
    def _acc():
        soft_ref[...] += psoft
        hard_ref[...] += phard

    @pl.when(pid == nblk - 1)
    def _fin():
        ssoft = soft_ref[...]
        shard = hard_ref[...]
        for j in range(NT // BC, N // BC):
            gpn_j = _gpn_from_gates(gates_ref[:, pl.ds(j * BC, BC)], permw)
            ssoft += jnp.sum(gpn_j, axis=0, keepdims=True)
            shard += jnp.sum(jnp.where(gpn_j < 1e-5, 0.0, 1.0), axis=0,
                             keepdims=True)
        soft_ref[...] = ssoft * (1.0 / N)
        hard_ref[...] = shard * (1.0 / N)


def _sc_combine_body(h_hbm, gexp_hbm, ysc_hbm, gexp_v, hb, yv,
                     hsem0, hsem1, ysem0, ysem1):
    hsem = [hsem0, hsem1]
    ysem = [ysem0, ysem1]
    info = plsc.get_sparse_core_info()
    nc, ns, nl = info.num_cores, info.num_subcores, info.num_lanes
    nw = nc * ns
    tw = NSC // nw
    nsb = tw // TB
    wid = lax.axis_index("s") * nc + lax.axis_index("c")
    wbase = wid * tw
    pltpu.sync_copy(gexp_hbm.at[pl.ds(wbase, tw), :], gexp_v)

    def in_slice(s):
        return h_hbm.at[:, pl.ds(NT + wbase + s * TB, TB), :]

    def start_in(s, b):
        pltpu.async_copy(in_slice(s), hb.at[b], hsem[b])

    def wait_in(s, b):
        pltpu.make_async_copy(in_slice(s), hb.at[b], hsem[b]).wait()

    def out_slice(s):
        return ysc_hbm.at[pl.ds(wbase + s * TB, TB), :]

    start_in(0, 0)
    start_in(1, 1)

    def step(i, _):
        for b in range(2):
            s = i * 2 + b
            wait_in(s, b)

            @pl.when(s >= 2)
            def _drain_y():
                pltpu.make_async_copy(out_slice(s - 2), yv.at[b],
                                      ysem[b]).wait()

            for t in range(TB):
                tloc = s * TB + t
                spl = [gexp_v[tloc, pl.ds(e * nl, nl)] for e in range(E)]

                def chunk(c4, _c):
                    for k in range(4):
                        off = c4 * (4 * nl) + k * nl
                        a = spl[0] * hb[b, 0, t, pl.ds(off, nl)]
                        for e in range(1, E):
                            a = a + spl[e] * hb[b, e, t, pl.ds(off, nl)]
                        yv[b, t, pl.ds(off, nl)] = a
                    return 0

                lax.fori_loop(0, D // (4 * nl), chunk, 0)

            pltpu.async_copy(yv.at[b], out_slice(s), ysem[b])

            @pl.when(s + 2 < nsb)
            def _next_in():
                start_in(s + 2, b)
        return 0

    lax.fori_loop(0, nsb // 2, step, 0)
    for b in range(2):
        pltpu.make_async_copy(out_slice(nsb - 2 + b), yv.at[b],
                              ysem[b]).wait()


def _merge_body(yfull_ref, ysc_ref, y_ref):
    y_ref[...] = ysc_ref[...]


def kernel(h, x, permutation_weights, W, b):
    b2 = b.reshape(E, 1)

    logits_t, permw = pl.pallas_call(
        _logits_body,
        grid=(N // BA,),
        in_specs=[
            pl.BlockSpec((BA, D), lambda i: (i, 0)),
            pl.BlockSpec((E, D), lambda i: (0, 0)),
            pl.BlockSpec((E, 1), lambda i: (0, 0)),
            pl.BlockSpec((P, E, E), lambda i: (0, 0, 0)),
        ],
        out_specs=[
            pl.BlockSpec((E, BA), lambda i: (0, i)),
            pl.BlockSpec((E, E), lambda i: (0, 0)),
        ],
        out_shape=[
            jax.ShapeDtypeStruct((E, N), jnp.float32),
            jax.ShapeDtypeStruct((E, E), jnp.float32),
        ],
    )(x, W, b2, permutation_weights)

    info = plsc.get_sparse_core_info()
    nw = info.num_cores * info.num_subcores
    tpw = N // nw
    gates = pl.kernel(
        _gate_sc_body,
        out_type=jax.ShapeDtypeStruct((E, N), jnp.float32),
        mesh=plsc.VectorSubcoreMesh(core_axis_name="c", subcore_axis_name="s"),
        scratch_types=[
            pltpu.VMEM((E, tpw), jnp.float32),
            pltpu.VMEM((E, tpw), jnp.float32),
        ],
    )(logits_t)

    gexp = pl.pallas_call(
        _gexp_body,
        in_specs=[
            pl.BlockSpec((E, N), lambda: (0, 0)),
            pl.BlockSpec((E, E), lambda: (0, 0)),
        ],
        out_specs=pl.BlockSpec((NSC, 128), lambda: (0, 0)),
        out_shape=jax.ShapeDtypeStruct((NSC, 128), jnp.float32),
    )(gates, permw)

    tw = NSC // nw
    ysc = pl.kernel(
        _sc_combine_body,
        out_type=jax.ShapeDtypeStruct((NSC, D), jnp.float32),
        mesh=plsc.VectorSubcoreMesh(core_axis_name="c", subcore_axis_name="s"),
        scratch_types=[
            pltpu.VMEM((tw, 128), jnp.float32),
            pltpu.VMEM((2, E, TB, D), jnp.float32),
            pltpu.VMEM((2, TB, D), jnp.float32),
            pltpu.SemaphoreType.DMA,
            pltpu.SemaphoreType.DMA,
            pltpu.SemaphoreType.DMA,
            pltpu.SemaphoreType.DMA,
        ],
    )(h, gexp)

    y_part, soft, hard = pl.pallas_call(
        _combine_body,
        grid=(NT // BC,),
        in_specs=[
            pl.BlockSpec((E, N), lambda i: (0, 0)),
            pl.BlockSpec((E, BC, D), lambda i: (0, i, 0)),
            pl.BlockSpec((E, E), lambda i: (0, 0)),
        ],
        out_specs=[
            pl.BlockSpec((BC, D), lambda i: (i, 0)),
            pl.BlockSpec((1, E), lambda i: (0, 0)),
            pl.BlockSpec((1, E), lambda i: (0, 0)),
        ],
        out_shape=[
            jax.ShapeDtypeStruct((N, D), jnp.float32),
            jax.ShapeDtypeStruct((1, E), jnp.float32),
            jax.ShapeDtypeStruct((1, E), jnp.float32),
        ],
    )(gates, h, permw)

    y = pl.pallas_call(
        _merge_body,
        grid=(NSC // BC,),
        in_specs=[
            pl.BlockSpec((BC, D), lambda i: (0, 0)),
            pl.BlockSpec((BC, D), lambda i: (i, 0)),
        ],
        out_specs=pl.BlockSpec((BC, D), lambda i: (NT // BC + i, 0)),
        out_shape=jax.ShapeDtypeStruct((N, D), jnp.float32),
        input_output_aliases={0: 0},
    )(y_part, ysc)

    return (y, soft.reshape(E, 1), hard.reshape(E, 1))


# speedup vs baseline: 1.4465x; 1.0845x over previous
import functools

import jax
import jax.numpy as jnp
from jax import lax
from jax.experimental import pallas as pl
from jax.experimental.pallas import tpu as pltpu
from jax.experimental.pallas import tpu_sc as plsc

E = 8
N = 4096
D = 1024
P = 4

BA = 512
BC = 256


def _logits_body(x_ref, w_ref, b_ref, perm_ref, lt_ref, permw_ref):
    lt = lax.dot_general(
        w_ref[...], x_ref[...],
        dimension_numbers=(((1,), (1,)), ((), ())),
        preferred_element_type=jnp.float32,
    )
    lt_ref[...] = lt + b_ref[...]
    permw_ref[...] = jnp.mean(perm_ref[...], axis=0)


def _gate_sc_body(lt_hbm, gates_hbm, lg_v, gates_v):
    info = plsc.get_sparse_core_info()
    nc, ns, nl = info.num_cores, info.num_subcores, info.num_lanes
    tpw = N // (nc * ns)
    wid = lax.axis_index("s") * nc + lax.axis_index("c")
    base = wid * tpw
    pltpu.sync_copy(lt_hbm.at[:, pl.ds(base, tpw)], lg_v)
    lane = lax.iota(jnp.int32, nl)
    for g in range(tpw // nl):
        l = [lg_v[e, pl.ds(g * nl, nl)] for e in range(E)]
        best = l[0]
        i1 = jnp.zeros((nl,), jnp.int32)
        for e in range(1, E):
            c = l[e] > best
            best = jnp.where(c, l[e], best)
            i1 = jnp.where(c, e, i1)
        best2 = jnp.full((nl,), -jnp.inf, jnp.float32)
        i2 = jnp.zeros((nl,), jnp.int32)
        for e in range(E):
            c = (i1 != e) & (l[e] > best2)
            best2 = jnp.where(c, l[e], best2)
            i2 = jnp.where(c, e, i2)
        p = []
        for e in range(E):
            sel = ((i1 == e) | (i2 == e)) & (l[e] != 0.0)
            p.append(jnp.where(sel, jnp.exp(l[e] - best), 0.0))
        denom = p[0]
        for e in range(1, E):
            denom = denom + p[e]
        inv = 1.0 / denom
        for e in range(E):
            gates_v[e, pl.ds(g * nl, nl)] = p[e] * inv
    pltpu.sync_copy(gates_v, gates_hbm.at[:, pl.ds(base, tpw)])


def _combine_body(gates_ref, h_ref, permw_ref, y_ref, soft_ref, hard_ref):
    pid = pl.program_id(0)
    nblk = pl.num_programs(0)
    g = jnp.transpose(gates_ref[...])
    gp = lax.dot_general(
        g, permw_ref[...],
        dimension_numbers=(((1,), (0,)), ((), ())),
        preferred_element_type=jnp.float32,
    )
    gpn = gp / jnp.sum(gp, axis=1, keepdims=True)
    acc = gpn[:, 0:1] * h_ref[0]
    for e in range(1, E):
        acc = acc + gpn[:, e:e + 1] * h_ref[e]
    y_ref[...] = acc
    psoft = jnp.sum(gpn, axis=0, keepdims=True)
    phard = jnp.sum(jnp.where(gpn < 1e-5, 0.0, 1.0), axis=0, keepdims=True)

    @pl.when(pid == 0)
    def _init():
        soft_ref[...] = psoft
        hard_ref[...] = phard

    @pl.when(pid != 0)
    def _acc():
        soft_ref[...] += psoft
        hard_ref[...] += phard

    @pl.when(pid == nblk - 1)
    def _fin():
        soft_ref[...] = soft_ref[...] * (1.0 / N)
        hard_ref[...] = hard_ref[...] * (1.0 / N)


def kernel(h, x, permutation_weights, W, b):
    b2 = b.reshape(E, 1)

    logits_t, permw = pl.pallas_call(
        _logits_body,
        grid=(N // BA,),
        in_specs=[
            pl.BlockSpec((BA, D), lambda i: (i, 0)),
            pl.BlockSpec((E, D), lambda i: (0, 0)),
            pl.BlockSpec((E, 1), lambda i: (0, 0)),
            pl.BlockSpec((P, E, E), lambda i: (0, 0, 0)),
        ],
        out_specs=[
            pl.BlockSpec((E, BA), lambda i: (0, i)),
            pl.BlockSpec((E, E), lambda i: (0, 0)),
        ],
        out_shape=[
            jax.ShapeDtypeStruct((E, N), jnp.float32),
            jax.ShapeDtypeStruct((E, E), jnp.float32),
        ],
    )(x, W, b2, permutation_weights)

    info = plsc.get_sparse_core_info()
    tpw = N // (info.num_cores * info.num_subcores)
    gates = pl.kernel(
        _gate_sc_body,
        out_type=jax.ShapeDtypeStruct((E, N), jnp.float32),
        mesh=plsc.VectorSubcoreMesh(core_axis_name="c", subcore_axis_name="s"),
        scratch_types=[
            pltpu.VMEM((E, tpw), jnp.float32),
            pltpu.VMEM((E, tpw), jnp.float32),
        ],
    )(logits_t)

    y, soft, hard = pl.pallas_call(
        _combine_body,
        grid=(N // BC,),
        in_specs=[
            pl.BlockSpec((E, BC), lambda i: (0, i)),
            pl.BlockSpec((E, BC, D), lambda i: (0, i, 0)),
            pl.BlockSpec((E, E), lambda i: (0, 0)),
        ],
        out_specs=[
            pl.BlockSpec((BC, D), lambda i: (i, 0)),
            pl.BlockSpec((1, E), lambda i: (0, 0)),
            pl.BlockSpec((1, E), lambda i: (0, 0)),
        ],
        out_shape=[
            jax.ShapeDtypeStruct((N, D), jnp.float32),
            jax.ShapeDtypeStruct((1, E), jnp.float32),
            jax.ShapeDtypeStruct((1, E), jnp.float32),
        ],
    )(gates, h, permw)

    return (y, soft.reshape(E, 1), hard.reshape(E, 1))


# speedup vs baseline: 1.4784x; 1.0221x over previous
import functools

import jax
import jax.numpy as jnp
from jax import lax
from jax.experimental import pallas as pl
from jax.experimental.pallas import tpu as pltpu
from jax.experimental.pallas import tpu_sc as plsc

E = 8
N = 4096
D = 1024
P = 4

BA = 1024
BC = 256


def _logits_body(x_ref, w_ref, b_ref, perm_ref, lt_ref, permw_ref):
    lt = lax.dot_general(
        w_ref[...], x_ref[...],
        dimension_numbers=(((1,), (1,)), ((), ())),
        preferred_element_type=jnp.float32,
    )
    lt_ref[...] = lt + b_ref[...]
    permw_ref[...] = jnp.mean(perm_ref[...], axis=0)


def _gate_sc_body(lt_hbm, gates_hbm, lg_v, gates_v):
    info = plsc.get_sparse_core_info()
    nc, ns, nl = info.num_cores, info.num_subcores, info.num_lanes
    tpw = N // (nc * ns)
    wid = lax.axis_index("s") * nc + lax.axis_index("c")
    base = wid * tpw
    pltpu.sync_copy(lt_hbm.at[:, pl.ds(base, tpw)], lg_v)
    lane = lax.iota(jnp.int32, nl)
    for g in range(tpw // nl):
        l = [lg_v[e, pl.ds(g * nl, nl)] for e in range(E)]
        best = l[0]
        i1 = jnp.zeros((nl,), jnp.int32)
        for e in range(1, E):
            c = l[e] > best
            best = jnp.where(c, l[e], best)
            i1 = jnp.where(c, e, i1)
        best2 = jnp.full((nl,), -jnp.inf, jnp.float32)
        i2 = jnp.zeros((nl,), jnp.int32)
        for e in range(E):
            c = (i1 != e) & (l[e] > best2)
            best2 = jnp.where(c, l[e], best2)
            i2 = jnp.where(c, e, i2)
        p = []
        for e in range(E):
            sel = ((i1 == e) | (i2 == e)) & (l[e] != 0.0)
            p.append(jnp.where(sel, jnp.exp(l[e] - best), 0.0))
        denom = p[0]
        for e in range(1, E):
            denom = denom + p[e]
        inv = 1.0 / denom
        for e in range(E):
            gates_v[e, pl.ds(g * nl, nl)] = p[e] * inv
    pltpu.sync_copy(gates_v, gates_hbm.at[:, pl.ds(base, tpw)])


def _combine_body(gates_ref, h_ref, permw_ref, y_ref, soft_ref, hard_ref):
    pid = pl.program_id(0)
    nblk = pl.num_programs(0)
    g = jnp.transpose(gates_ref[...])
    gp = lax.dot_general(
        g, permw_ref[...],
        dimension_numbers=(((1,), (0,)), ((), ())),
        preferred_element_type=jnp.float32,
    )
    gpn = gp / jnp.sum(gp, axis=1, keepdims=True)
    acc = gpn[:, 0:1] * h_ref[0]
    for e in range(1, E):
        acc = acc + gpn[:, e:e + 1] * h_ref[e]
    y_ref[...] = acc
    psoft = jnp.sum(gpn, axis=0, keepdims=True)
    phard = jnp.sum(jnp.where(gpn < 1e-5, 0.0, 1.0), axis=0, keepdims=True)

    @pl.when(pid == 0)
    def _init():
        soft_ref[...] = psoft
        hard_ref[...] = phard

    @pl.when(pid != 0)
    def _acc():
        soft_ref[...] += psoft
        hard_ref[...] += phard

    @pl.when(pid == nblk - 1)
    def _fin():
        soft_ref[...] = soft_ref[...] * (1.0 / N)
        hard_ref[...] = hard_ref[...] * (1.0 / N)


def kernel(h, x, permutation_weights, W, b):
    b2 = b.reshape(E, 1)

    logits_t, permw = pl.pallas_call(
        _logits_body,
        grid=(N // BA,),
        in_specs=[
            pl.BlockSpec((BA, D), lambda i: (i, 0)),
            pl.BlockSpec((E, D), lambda i: (0, 0)),
            pl.BlockSpec((E, 1), lambda i: (0, 0)),
            pl.BlockSpec((P, E, E), lambda i: (0, 0, 0)),
        ],
        out_specs=[
            pl.BlockSpec((E, BA), lambda i: (0, i)),
            pl.BlockSpec((E, E), lambda i: (0, 0)),
        ],
        out_shape=[
            jax.ShapeDtypeStruct((E, N), jnp.float32),
            jax.ShapeDtypeStruct((E, E), jnp.float32),
        ],
    )(x, W, b2, permutation_weights)

    info = plsc.get_sparse_core_info()
    tpw = N // (info.num_cores * info.num_subcores)
    gates = pl.kernel(
        _gate_sc_body,
        out_type=jax.ShapeDtypeStruct((E, N), jnp.float32),
        mesh=plsc.VectorSubcoreMesh(core_axis_name="c", subcore_axis_name="s"),
        scratch_types=[
            pltpu.VMEM((E, tpw), jnp.float32),
            pltpu.VMEM((E, tpw), jnp.float32),
        ],
    )(logits_t)

    y, soft, hard = pl.pallas_call(
        _combine_body,
        grid=(N // BC,),
        in_specs=[
            pl.BlockSpec((E, BC), lambda i: (0, i)),
            pl.BlockSpec((E, BC, D), lambda i: (0, i, 0)),
            pl.BlockSpec((E, E), lambda i: (0, 0)),
        ],
        out_specs=[
            pl.BlockSpec((BC, D), lambda i: (i, 0)),
            pl.BlockSpec((1, E), lambda i: (0, 0)),
            pl.BlockSpec((1, E), lambda i: (0, 0)),
        ],
        out_shape=[
            jax.ShapeDtypeStruct((N, D), jnp.float32),
            jax.ShapeDtypeStruct((1, E), jnp.float32),
            jax.ShapeDtypeStruct((1, E), jnp.float32),
        ],
    )(gates, h, permw)

    return (y, soft.reshape(E, 1), hard.reshape(E, 1))


# speedup vs baseline: 1.4860x; 1.0051x over previous
import functools

import jax
import jax.numpy as jnp
from jax import lax
from jax.experimental import pallas as pl
from jax.experimental.pallas import tpu as pltpu
from jax.experimental.pallas import tpu_sc as plsc

E = 8
N = 4096
D = 1024
P = 4

BA = 2048
BC = 256


def _logits_body(x_ref, w_ref, b_ref, perm_ref, lt_ref, permw_ref):
    lt = lax.dot_general(
        w_ref[...], x_ref[...],
        dimension_numbers=(((1,), (1,)), ((), ())),
        preferred_element_type=jnp.float32,
    )
    lt_ref[...] = lt + b_ref[...]
    permw_ref[...] = jnp.mean(perm_ref[...], axis=0)


def _gate_sc_body(lt_hbm, gates_hbm, lg_v, gates_v):
    info = plsc.get_sparse_core_info()
    nc, ns, nl = info.num_cores, info.num_subcores, info.num_lanes
    tpw = N // (nc * ns)
    wid = lax.axis_index("s") * nc + lax.axis_index("c")
    base = wid * tpw
    pltpu.sync_copy(lt_hbm.at[:, pl.ds(base, tpw)], lg_v)
    lane = lax.iota(jnp.int32, nl)
    for g in range(tpw // nl):
        l = [lg_v[e, pl.ds(g * nl, nl)] for e in range(E)]
        best = l[0]
        i1 = jnp.zeros((nl,), jnp.int32)
        for e in range(1, E):
            c = l[e] > best
            best = jnp.where(c, l[e], best)
            i1 = jnp.where(c, e, i1)
        best2 = jnp.full((nl,), -jnp.inf, jnp.float32)
        i2 = jnp.zeros((nl,), jnp.int32)
        for e in range(E):
            c = (i1 != e) & (l[e] > best2)
            best2 = jnp.where(c, l[e], best2)
            i2 = jnp.where(c, e, i2)
        p = []
        for e in range(E):
            sel = ((i1 == e) | (i2 == e)) & (l[e] != 0.0)
            p.append(jnp.where(sel, jnp.exp(l[e] - best), 0.0))
        denom = p[0]
        for e in range(1, E):
            denom = denom + p[e]
        inv = 1.0 / denom
        for e in range(E):
            gates_v[e, pl.ds(g * nl, nl)] = p[e] * inv
    pltpu.sync_copy(gates_v, gates_hbm.at[:, pl.ds(base, tpw)])


def _combine_body(gates_ref, h_ref, permw_ref, y_ref, soft_ref, hard_ref):
    pid = pl.program_id(0)
    nblk = pl.num_programs(0)
    g = jnp.transpose(gates_ref[...])
    gp = lax.dot_general(
        g, permw_ref[...],
        dimension_numbers=(((1,), (0,)), ((), ())),
        preferred_element_type=jnp.float32,
    )
    gpn = gp / jnp.sum(gp, axis=1, keepdims=True)
    acc = gpn[:, 0:1] * h_ref[0]
    for e in range(1, E):
        acc = acc + gpn[:, e:e + 1] * h_ref[e]
    y_ref[...] = acc
    psoft = jnp.sum(gpn, axis=0, keepdims=True)
    phard = jnp.sum(jnp.where(gpn < 1e-5, 0.0, 1.0), axis=0, keepdims=True)

    @pl.when(pid == 0)
    def _init():
        soft_ref[...] = psoft
        hard_ref[...] = phard

    @pl.when(pid != 0)
    def _acc():
        soft_ref[...] += psoft
        hard_ref[...] += phard

    @pl.when(pid == nblk - 1)
    def _fin():
        soft_ref[...] = soft_ref[...] * (1.0 / N)
        hard_ref[...] = hard_ref[...] * (1.0 / N)


def kernel(h, x, permutation_weights, W, b):
    b2 = b.reshape(E, 1)

    logits_t, permw = pl.pallas_call(
        _logits_body,
        grid=(N // BA,),
        in_specs=[
            pl.BlockSpec((BA, D), lambda i: (i, 0)),
            pl.BlockSpec((E, D), lambda i: (0, 0)),
            pl.BlockSpec((E, 1), lambda i: (0, 0)),
            pl.BlockSpec((P, E, E), lambda i: (0, 0, 0)),
        ],
        out_specs=[
            pl.BlockSpec((E, BA), lambda i: (0, i)),
            pl.BlockSpec((E, E), lambda i: (0, 0)),
        ],
        out_shape=[
            jax.ShapeDtypeStruct((E, N), jnp.float32),
            jax.ShapeDtypeStruct((E, E), jnp.float32),
        ],
    )(x, W, b2, permutation_weights)

    info = plsc.get_sparse_core_info()
    tpw = N // (info.num_cores * info.num_subcores)
    gates = pl.kernel(
        _gate_sc_body,
        out_type=jax.ShapeDtypeStruct((E, N), jnp.float32),
        mesh=plsc.VectorSubcoreMesh(core_axis_name="c", subcore_axis_name="s"),
        scratch_types=[
            pltpu.VMEM((E, tpw), jnp.float32),
            pltpu.VMEM((E, tpw), jnp.float32),
        ],
    )(logits_t)

    y, soft, hard = pl.pallas_call(
        _combine_body,
        grid=(N // BC,),
        in_specs=[
            pl.BlockSpec((E, BC), lambda i: (0, i)),
            pl.BlockSpec((E, BC, D), lambda i: (0, i, 0)),
            pl.BlockSpec((E, E), lambda i: (0, 0)),
        ],
        out_specs=[
            pl.BlockSpec((BC, D), lambda i: (i, 0)),
            pl.BlockSpec((1, E), lambda i: (0, 0)),
            pl.BlockSpec((1, E), lambda i: (0, 0)),
        ],
        out_shape=[
            jax.ShapeDtypeStruct((N, D), jnp.float32),
            jax.ShapeDtypeStruct((1, E), jnp.float32),
            jax.ShapeDtypeStruct((1, E), jnp.float32),
        ],
    )(gates, h, permw)

    return (y, soft.reshape(E, 1), hard.reshape(E, 1))
